# Initial kernel scaffold; baseline (speedup 1.0000x reference)
import functools

import jax
import jax.numpy as jnp
from jax import lax
from jax.experimental import pallas as pl
from jax.experimental.pallas import tpu as pltpu
from jax.experimental.pallas import tpu_sc as plsc

N = 10000
NP = 10240
D = 128
H = 16
NC = 2
NS = 16
NW = NC * NS
E = 320000
EP = E // NW
CH = 128
K1 = (EP + CH - 1) // CH
EPP = K1 * CH
RPT = NP // NS

_mesh = plsc.VectorSubcoreMesh(core_axis_name="c", subcore_axis_name="s")


@functools.partial(
    pl.kernel,
    out_type=jax.ShapeDtypeStruct((NW, NP), jnp.float32),
    mesh=_mesh,
    scratch_types=[
        pltpu.VMEM((EP,), jnp.int32),
        pltpu.VMEM((NP,), jnp.float32),
        pltpu.SemaphoreType.DMA,
    ],
)
def _deg_kernel(dst_hbm, out_hbm, dst_v, acc_v, sem):
    del sem
    wid = lax.axis_index("s") * NC + lax.axis_index("c")
    pltpu.sync_copy(dst_hbm.at[wid], dst_v)

    def zero_body(i, carry):
        acc_v[pl.ds(i * 16, 16)] = jnp.zeros((16,), jnp.float32)
        return carry

    lax.fori_loop(0, NP // 16, zero_body, 0)
    ones = jnp.ones((16,), jnp.float32)

    def body(j, carry):
        d16 = dst_v[pl.ds(j * 16, 16)]
        plsc.addupdate_scatter(acc_v, [d16], ones)
        return carry

    lax.fori_loop(0, EP // 16, body, 0)
    pltpu.sync_copy(acc_v, out_hbm.at[wid])


@functools.partial(
    pl.kernel,
    out_type=jax.ShapeDtypeStruct((NC, NP, H), jnp.float32),
    mesh=_mesh,
    scratch_types=[
        pltpu.VMEM((K1, CH), jnp.int32),
        pltpu.VMEM((K1, CH), jnp.int32),
        pltpu.VMEM((CH, H), jnp.float32),
        pltpu.VMEM((RPT, H), jnp.float32),
        pltpu.VMEM_SHARED((NP, H), jnp.float32),
        pltpu.SemaphoreType.DMA,
    ],
)
def _agg1_kernel(hs_hbm, src_hbm, dst_hbm, out_hbm, si, di, rows, zbuf,
                 acc_sh, sem):
    cid = lax.axis_index("c")
    sid = lax.axis_index("s")
    wid = sid * NC + cid
    pltpu.sync_copy(src_hbm.at[wid], si)
    pltpu.sync_copy(dst_hbm.at[wid], di)

    def zero_body(i, carry):
        zbuf[i, :] = jnp.zeros((H,), jnp.float32)
        return carry

    lax.fori_loop(0, RPT, zero_body, 0)
    pltpu.sync_copy(zbuf, acc_sh.at[pl.ds(sid * RPT, RPT)])
    plsc.subcore_barrier()

    def body(j, carry):
        pltpu.async_copy(hs_hbm.at[si.at[j]], rows, sem).wait()
        pltpu.sync_copy(rows, acc_sh.at[di.at[j]], add=True)
        return carry

    lax.fori_loop(0, K1, body, 0)
    plsc.subcore_barrier()
    sl = pl.ds(sid * RPT, RPT)
    pltpu.sync_copy(acc_sh.at[sl], out_hbm.at[cid].at[sl])


@functools.partial(
    pl.kernel,
    out_type=jax.ShapeDtypeStruct((NW, NP), jnp.float32),
    mesh=_mesh,
    scratch_types=[
        pltpu.VMEM((NP,), jnp.float32),
        pltpu.VMEM((EP,), jnp.int32),
        pltpu.VMEM((EP,), jnp.int32),
        pltpu.VMEM((NP,), jnp.float32),
        pltpu.SemaphoreType.DMA,
    ],
)
def _agg2_kernel(hs_hbm, src_hbm, dst_hbm, out_hbm, hv, sv, dv, acc_v, sem):
    del sem
    wid = lax.axis_index("s") * NC + lax.axis_index("c")
    pltpu.sync_copy(hs_hbm, hv)
    pltpu.sync_copy(src_hbm.at[wid], sv)
    pltpu.sync_copy(dst_hbm.at[wid], dv)

    def zero_body(i, carry):
        acc_v[pl.ds(i * 16, 16)] = jnp.zeros((16,), jnp.float32)
        return carry

    lax.fori_loop(0, NP // 16, zero_body, 0)

    def body(j, carry):
        s16 = sv[pl.ds(j * 16, 16)]
        d16 = dv[pl.ds(j * 16, 16)]
        vals = plsc.load_gather(hv, [s16])
        plsc.addupdate_scatter(acc_v, [d16], vals)
        return carry

    lax.fori_loop(0, EP // 16, body, 0)
    pltpu.sync_copy(acc_v, out_hbm.at[wid])


_R = 2048


def _tc1_body(x_ref, w_ref, degt_ref, hs_ref, dis_ref):
    deg = 1.0 + jnp.sum(degt_ref[...], axis=1, keepdims=True)
    dis = lax.rsqrt(deg)
    h = jnp.dot(x_ref[...], w_ref[...], preferred_element_type=jnp.float32)
    hs_ref[...] = h * dis
    dis_ref[...] = dis


def _tc2_body(p_ref, hs_ref, dis_ref, w2_ref, b1_ref, hs2_ref):
    agg = p_ref[0] + p_ref[1] + hs_ref[...]
    dis = dis_ref[...]
    out1 = agg * dis + b1_ref[...]
    r = jnp.maximum(out1, 0.0)
    h2 = jnp.dot(r, w2_ref[...], preferred_element_type=jnp.float32)
    hs2_ref[...] = h2 * dis


def _tc3_body(p_ref, hs2_ref, dis_ref, b2_ref, out_ref):
    agg = jnp.sum(p_ref[...], axis=1, keepdims=True)
    out_ref[...] = dis_ref[...] * (agg + hs2_ref[...]) + b2_ref[...]


def kernel(x, edge_index, W1, b1, W2, b2):
    src = edge_index[0]
    dst = edge_index[1]
    xp = jnp.pad(x, ((0, NP - N), (0, 0)))
    src_w = src.reshape(NW, EP)
    dst_w = dst.reshape(NW, EP)

    padn = EPP - EP
    pad_idx = N + (jnp.arange(padn, dtype=jnp.int32) % (NP - N))
    pad_blk = jnp.broadcast_to(pad_idx, (NW, padn))
    src3 = jnp.concatenate([src_w, pad_blk], axis=1).reshape(NW, K1, CH)
    dst3 = jnp.concatenate([dst_w, pad_blk], axis=1).reshape(NW, K1, CH)

    degp = _deg_kernel(dst_w)
    degt = degp.T

    hs1, dis = pl.pallas_call(
        _tc1_body,
        grid=(NP // _R,),
        in_specs=[
            pl.BlockSpec((_R, D), lambda i: (i, 0)),
            pl.BlockSpec((D, H), lambda i: (0, 0)),
            pl.BlockSpec((_R, NW), lambda i: (i, 0)),
        ],
        out_specs=[
            pl.BlockSpec((_R, H), lambda i: (i, 0)),
            pl.BlockSpec((_R, 1), lambda i: (i, 0)),
        ],
        out_shape=[
            jax.ShapeDtypeStruct((NP, H), jnp.float32),
            jax.ShapeDtypeStruct((NP, 1), jnp.float32),
        ],
    )(xp, W1, degt)

    part1 = _agg1_kernel(hs1, src3, dst3)

    hs2 = pl.pallas_call(
        _tc2_body,
        grid=(NP // _R,),
        in_specs=[
            pl.BlockSpec((NC, _R, H), lambda i: (0, i, 0)),
            pl.BlockSpec((_R, H), lambda i: (i, 0)),
            pl.BlockSpec((_R, 1), lambda i: (i, 0)),
            pl.BlockSpec((H, 1), lambda i: (0, 0)),
            pl.BlockSpec((1, H), lambda i: (0, 0)),
        ],
        out_specs=pl.BlockSpec((_R, 1), lambda i: (i, 0)),
        out_shape=jax.ShapeDtypeStruct((NP, 1), jnp.float32),
    )(part1, hs1, dis, W2, b1.reshape(1, H))

    part2 = _agg2_kernel(hs2.reshape(NP), src_w, dst_w)
    part2t = part2.T

    out = pl.pallas_call(
        _tc3_body,
        grid=(NP // _R,),
        in_specs=[
            pl.BlockSpec((_R, NW), lambda i: (i, 0)),
            pl.BlockSpec((_R, 1), lambda i: (i, 0)),
            pl.BlockSpec((_R, 1), lambda i: (i, 0)),
            pl.BlockSpec((1, 1), lambda i: (0, 0)),
        ],
        out_specs=pl.BlockSpec((_R, 1), lambda i: (i, 0)),
        out_shape=jax.ShapeDtypeStruct((NP, 1), jnp.float32),
    )(part2t, hs2, dis, b2.reshape(1, 1))

    return out[:N]


# speedup vs baseline: 51.2883x; 51.2883x over previous
import functools

import jax
import jax.numpy as jnp
from jax import lax
from jax.experimental import pallas as pl
from jax.experimental.pallas import tpu as pltpu
from jax.experimental.pallas import tpu_sc as plsc

N = 10000
NP = 10240
D = 128
H = 16
NC = 2
NS = 16
NW = NC * NS
E = 320000
EP = E // NW
CH = 128
K1 = (EP + CH - 1) // CH
EPP = K1 * CH
RPT = NP // NS

_mesh = plsc.VectorSubcoreMesh(core_axis_name="c", subcore_axis_name="s")


@functools.partial(
    pl.kernel,
    out_type=jax.ShapeDtypeStruct((NW, NP), jnp.float32),
    mesh=_mesh,
    compiler_params=pltpu.CompilerParams(needs_layout_passes=False, use_tc_tiling_on_sc=False),
    scratch_types=[
        pltpu.VMEM((EP,), jnp.int32),
        pltpu.VMEM((NP,), jnp.float32),
        pltpu.SemaphoreType.DMA,
    ],
)
def _deg_kernel(dst_hbm, out_hbm, dst_v, acc_v, sem):
    del sem
    wid = lax.axis_index("s") * NC + lax.axis_index("c")
    pltpu.sync_copy(dst_hbm.at[wid], dst_v)

    def zero_body(i, carry):
        acc_v[pl.ds(i * 16, 16)] = jnp.zeros((16,), jnp.float32)
        return carry

    lax.fori_loop(0, NP // 16, zero_body, 0)
    ones = jnp.ones((16,), jnp.float32)

    def body(j, carry):
        d16 = dst_v[pl.ds(j * 16, 16)]
        plsc.addupdate_scatter(acc_v, [d16], ones)
        return carry

    lax.fori_loop(0, EP // 16, body, 0)
    pltpu.sync_copy(acc_v, out_hbm.at[wid])


@functools.partial(
    pl.kernel,
    out_type=jax.ShapeDtypeStruct((NC, NP, H), jnp.float32),
    mesh=_mesh,
    compiler_params=pltpu.CompilerParams(needs_layout_passes=False, use_tc_tiling_on_sc=False),
    scratch_types=[
        pltpu.VMEM((K1, CH), jnp.int32),
        pltpu.VMEM((K1, CH), jnp.int32),
        pltpu.VMEM((CH, H), jnp.float32),
        pltpu.VMEM((RPT, H), jnp.float32),
        pltpu.VMEM_SHARED((NP, H), jnp.float32),
        pltpu.SemaphoreType.DMA,
    ],
)
def _agg1_kernel(hs_hbm, src_hbm, dst_hbm, out_hbm, si, di, rows, zbuf,
                 acc_sh, sem):
    cid = lax.axis_index("c")
    sid = lax.axis_index("s")
    wid = sid * NC + cid
    pltpu.sync_copy(src_hbm.at[wid], si)
    pltpu.sync_copy(dst_hbm.at[wid], di)

    def zero_body(i, carry):
        zbuf[i, :] = jnp.zeros((H,), jnp.float32)
        return carry

    lax.fori_loop(0, RPT, zero_body, 0)
    pltpu.sync_copy(zbuf, acc_sh.at[pl.ds(sid * RPT, RPT)])
    plsc.subcore_barrier()

    def body(j, carry):
        pltpu.async_copy(hs_hbm.at[si.at[j]], rows, sem).wait()
        pltpu.sync_copy(rows, acc_sh.at[di.at[j]], add=True)
        return carry

    lax.fori_loop(0, K1, body, 0)
    plsc.subcore_barrier()
    sl = pl.ds(sid * RPT, RPT)
    pltpu.sync_copy(acc_sh.at[sl], out_hbm.at[cid].at[sl])


@functools.partial(
    pl.kernel,
    out_type=jax.ShapeDtypeStruct((NW, NP), jnp.float32),
    mesh=_mesh,
    compiler_params=pltpu.CompilerParams(needs_layout_passes=False, use_tc_tiling_on_sc=False),
    scratch_types=[
        pltpu.VMEM((NP,), jnp.float32),
        pltpu.VMEM((EP,), jnp.int32),
        pltpu.VMEM((EP,), jnp.int32),
        pltpu.VMEM((NP,), jnp.float32),
        pltpu.SemaphoreType.DMA,
    ],
)
def _agg2_kernel(hs_hbm, src_hbm, dst_hbm, out_hbm, hv, sv, dv, acc_v, sem):
    del sem
    wid = lax.axis_index("s") * NC + lax.axis_index("c")
    pltpu.sync_copy(hs_hbm, hv)
    pltpu.sync_copy(src_hbm.at[wid], sv)
    pltpu.sync_copy(dst_hbm.at[wid], dv)

    def zero_body(i, carry):
        acc_v[pl.ds(i * 16, 16)] = jnp.zeros((16,), jnp.float32)
        return carry

    lax.fori_loop(0, NP // 16, zero_body, 0)

    def body(j, carry):
        s16 = sv[pl.ds(j * 16, 16)]
        d16 = dv[pl.ds(j * 16, 16)]
        vals = plsc.load_gather(hv, [s16])
        plsc.addupdate_scatter(acc_v, [d16], vals)
        return carry

    lax.fori_loop(0, EP // 16, body, 0)
    pltpu.sync_copy(acc_v, out_hbm.at[wid])


_R = 2048


def _tc1_body(x_ref, w_ref, degt_ref, hs_ref, dis_ref):
    deg = 1.0 + jnp.sum(degt_ref[...], axis=1, keepdims=True)
    dis = lax.rsqrt(deg)
    h = jnp.dot(x_ref[...], w_ref[...], preferred_element_type=jnp.float32)
    hs_ref[...] = h * dis
    dis_ref[...] = dis


def _tc2_body(p_ref, hs_ref, dis_ref, w2_ref, b1_ref, hs2_ref):
    agg = p_ref[0] + p_ref[1] + hs_ref[...]
    dis = dis_ref[...]
    out1 = agg * dis + b1_ref[...]
    r = jnp.maximum(out1, 0.0)
    h2 = jnp.dot(r, w2_ref[...], preferred_element_type=jnp.float32)
    hs2_ref[...] = h2 * dis


def _tc3_body(p_ref, hs2_ref, dis_ref, b2_ref, out_ref):
    agg = jnp.sum(p_ref[...], axis=1, keepdims=True)
    out_ref[...] = dis_ref[...] * (agg + hs2_ref[...]) + b2_ref[...]


def kernel(x, edge_index, W1, b1, W2, b2):
    src = edge_index[0]
    dst = edge_index[1]
    xp = jnp.pad(x, ((0, NP - N), (0, 0)))
    src_w = src.reshape(NW, EP)
    dst_w = dst.reshape(NW, EP)

    padn = EPP - EP
    pad_idx = N + (jnp.arange(padn, dtype=jnp.int32) % (NP - N))
    pad_blk = jnp.broadcast_to(pad_idx, (NW, padn))
    src3 = jnp.concatenate([src_w, pad_blk], axis=1).reshape(NW, K1, CH)
    dst3 = jnp.concatenate([dst_w, pad_blk], axis=1).reshape(NW, K1, CH)

    degp = _deg_kernel(dst_w)
    degt = degp.T

    hs1, dis = pl.pallas_call(
        _tc1_body,
        grid=(NP // _R,),
        in_specs=[
            pl.BlockSpec((_R, D), lambda i: (i, 0)),
            pl.BlockSpec((D, H), lambda i: (0, 0)),
            pl.BlockSpec((_R, NW), lambda i: (i, 0)),
        ],
        out_specs=[
            pl.BlockSpec((_R, H), lambda i: (i, 0)),
            pl.BlockSpec((_R, 1), lambda i: (i, 0)),
        ],
        out_shape=[
            jax.ShapeDtypeStruct((NP, H), jnp.float32),
            jax.ShapeDtypeStruct((NP, 1), jnp.float32),
        ],
    )(xp, W1, degt)

    part1 = _agg1_kernel(hs1, src3, dst3)

    hs2 = pl.pallas_call(
        _tc2_body,
        grid=(NP // _R,),
        in_specs=[
            pl.BlockSpec((NC, _R, H), lambda i: (0, i, 0)),
            pl.BlockSpec((_R, H), lambda i: (i, 0)),
            pl.BlockSpec((_R, 1), lambda i: (i, 0)),
            pl.BlockSpec((H, 1), lambda i: (0, 0)),
            pl.BlockSpec((1, H), lambda i: (0, 0)),
        ],
        out_specs=pl.BlockSpec((_R, 1), lambda i: (i, 0)),
        out_shape=jax.ShapeDtypeStruct((NP, 1), jnp.float32),
    )(part1, hs1, dis, W2, b1.reshape(1, H))

    part2 = _agg2_kernel(hs2.reshape(NP), src_w, dst_w)
    part2t = part2.T

    out = pl.pallas_call(
        _tc3_body,
        grid=(NP // _R,),
        in_specs=[
            pl.BlockSpec((_R, NW), lambda i: (i, 0)),
            pl.BlockSpec((_R, 1), lambda i: (i, 0)),
            pl.BlockSpec((_R, 1), lambda i: (i, 0)),
            pl.BlockSpec((1, 1), lambda i: (0, 0)),
        ],
        out_specs=pl.BlockSpec((_R, 1), lambda i: (i, 0)),
        out_shape=jax.ShapeDtypeStruct((NP, 1), jnp.float32),
    )(part2t, hs2, dis, b2.reshape(1, 1))

    return out[:N]


# speedup vs baseline: 70.2353x; 1.3694x over previous
import functools

import jax
import jax.numpy as jnp
from jax import lax
from jax.experimental import pallas as pl
from jax.experimental.pallas import tpu as pltpu
from jax.experimental.pallas import tpu_sc as plsc

N = 10000
NP = 10240
D = 128
H = 16
NC = 2
NS = 16
NW = NC * NS
E = 320000
EP = E // NW
CH = 128
G = 8
K1 = 80
NG = K1 // G
EPP = K1 * CH
RPT = NP // NS
U = 5

_mesh = plsc.VectorSubcoreMesh(core_axis_name="c", subcore_axis_name="s")


@functools.partial(
    pl.kernel,
    out_type=jax.ShapeDtypeStruct((NW, NP), jnp.float32),
    mesh=_mesh,
    compiler_params=pltpu.CompilerParams(needs_layout_passes=False, use_tc_tiling_on_sc=False),
    scratch_types=[
        pltpu.VMEM((EP,), jnp.int32),
        pltpu.VMEM((NP,), jnp.float32),
        pltpu.SemaphoreType.DMA,
    ],
)
def _deg_kernel(dst_hbm, out_hbm, dst_v, acc_v, sem):
    del sem
    wid = lax.axis_index("s") * NC + lax.axis_index("c")
    pltpu.sync_copy(dst_hbm.at[wid], dst_v)

    def zero_body(i, carry):
        acc_v[pl.ds(i * 16, 16)] = jnp.zeros((16,), jnp.float32)
        return carry

    lax.fori_loop(0, NP // 16, zero_body, 0)
    ones = jnp.ones((16,), jnp.float32)

    def body(j, carry):
        for u in range(U):
            d16 = dst_v[pl.ds((j * U + u) * 16, 16)]
            plsc.addupdate_scatter(acc_v, [d16], ones)
        return carry

    lax.fori_loop(0, EP // (16 * U), body, 0)
    pltpu.sync_copy(acc_v, out_hbm.at[wid])


@functools.partial(
    pl.kernel,
    out_type=jax.ShapeDtypeStruct((NC, NP, H), jnp.float32),
    mesh=_mesh,
    compiler_params=pltpu.CompilerParams(needs_layout_passes=False, use_tc_tiling_on_sc=False),
    scratch_types=[
        pltpu.VMEM((K1, CH), jnp.int32),
        pltpu.VMEM((K1, CH), jnp.int32),
        pltpu.VMEM((2, G, CH, H), jnp.float32),
        pltpu.VMEM((RPT, H), jnp.float32),
        pltpu.VMEM_SHARED((NP, H), jnp.float32),
        pltpu.SemaphoreType.DMA,
    ],
)
def _agg1_kernel(hs_hbm, src_hbm, dst_hbm, out_hbm, si, di, rows, zbuf,
                 acc_sh, sem):
    cid = lax.axis_index("c")
    sid = lax.axis_index("s")
    wid = sid * NC + cid
    pltpu.sync_copy(src_hbm.at[wid], si)
    pltpu.sync_copy(dst_hbm.at[wid], di)

    def zero_body(i, carry):
        zbuf[i, :] = jnp.zeros((H,), jnp.float32)
        return carry

    lax.fori_loop(0, RPT, zero_body, 0)
    pltpu.sync_copy(zbuf, acc_sh.at[pl.ds(sid * RPT, RPT)])
    plsc.subcore_barrier()

    for b in range(G):
        pltpu.async_copy(hs_hbm.at[si.at[b]], rows.at[0, b], sem)

    def body(g, carry):
        pbuf = lax.rem(g, 2)
        nbuf = lax.rem(g + 1, 2)

        @pl.when(g + 1 < NG)
        def _fire():
            for b in range(G):
                pltpu.async_copy(hs_hbm.at[si.at[(g + 1) * G + b]],
                                 rows.at[nbuf, b], sem)

        for b in range(G):
            pltpu.make_async_copy(hs_hbm.at[si.at[g * G + b]],
                                  rows.at[pbuf, b], sem).wait()
        for b in range(G):
            pltpu.sync_copy(rows.at[pbuf, b], acc_sh.at[di.at[g * G + b]],
                            add=True)
        return carry

    lax.fori_loop(0, NG, body, 0)
    plsc.subcore_barrier()
    sl = pl.ds(sid * RPT, RPT)
    pltpu.sync_copy(acc_sh.at[sl], out_hbm.at[cid].at[sl])


@functools.partial(
    pl.kernel,
    out_type=jax.ShapeDtypeStruct((NW, NP), jnp.float32),
    mesh=_mesh,
    compiler_params=pltpu.CompilerParams(needs_layout_passes=False, use_tc_tiling_on_sc=False),
    scratch_types=[
        pltpu.VMEM((NP,), jnp.float32),
        pltpu.VMEM((EP,), jnp.int32),
        pltpu.VMEM((EP,), jnp.int32),
        pltpu.VMEM((NP,), jnp.float32),
        pltpu.SemaphoreType.DMA,
    ],
)
def _agg2_kernel(hs_hbm, src_hbm, dst_hbm, out_hbm, hv, sv, dv, acc_v, sem):
    del sem
    wid = lax.axis_index("s") * NC + lax.axis_index("c")
    pltpu.sync_copy(hs_hbm, hv)
    pltpu.sync_copy(src_hbm.at[wid], sv)
    pltpu.sync_copy(dst_hbm.at[wid], dv)

    def zero_body(i, carry):
        acc_v[pl.ds(i * 16, 16)] = jnp.zeros((16,), jnp.float32)
        return carry

    lax.fori_loop(0, NP // 16, zero_body, 0)

    def body(j, carry):
        for u in range(U):
            s16 = sv[pl.ds((j * U + u) * 16, 16)]
            d16 = dv[pl.ds((j * U + u) * 16, 16)]
            vals = plsc.load_gather(hv, [s16])
            plsc.addupdate_scatter(acc_v, [d16], vals)
        return carry

    lax.fori_loop(0, EP // (16 * U), body, 0)
    pltpu.sync_copy(acc_v, out_hbm.at[wid])


_R = 2048


def _tc1_body(x_ref, w_ref, degt_ref, hs_ref, dis_ref):
    deg = 1.0 + jnp.sum(degt_ref[...], axis=1, keepdims=True)
    dis = lax.rsqrt(deg)
    h = jnp.dot(x_ref[...], w_ref[...], preferred_element_type=jnp.float32)
    hs_ref[...] = h * dis
    dis_ref[...] = dis


def _tc2_body(p_ref, hs_ref, dis_ref, w2_ref, b1_ref, hs2_ref):
    agg = p_ref[0] + p_ref[1] + hs_ref[...]
    dis = dis_ref[...]
    out1 = agg * dis + b1_ref[...]
    r = jnp.maximum(out1, 0.0)
    h2 = jnp.dot(r, w2_ref[...], preferred_element_type=jnp.float32)
    hs2_ref[...] = h2 * dis


def _tc3_body(p_ref, hs2_ref, dis_ref, b2_ref, out_ref):
    agg = jnp.sum(p_ref[...], axis=1, keepdims=True)
    out_ref[...] = dis_ref[...] * (agg + hs2_ref[...]) + b2_ref[...]


def kernel(x, edge_index, W1, b1, W2, b2):
    src = edge_index[0]
    dst = edge_index[1]
    xp = jnp.pad(x, ((0, NP - N), (0, 0)))
    src_w = src.reshape(NW, EP)
    dst_w = dst.reshape(NW, EP)

    padn = EPP - EP
    pad_idx = N + (jnp.arange(padn, dtype=jnp.int32) % (NP - N))
    pad_blk = jnp.broadcast_to(pad_idx, (NW, padn))
    src3 = jnp.concatenate([src_w, pad_blk], axis=1).reshape(NW, K1, CH)
    dst3 = jnp.concatenate([dst_w, pad_blk], axis=1).reshape(NW, K1, CH)

    degp = _deg_kernel(dst_w)
    degt = degp.T

    hs1, dis = pl.pallas_call(
        _tc1_body,
        grid=(NP // _R,),
        in_specs=[
            pl.BlockSpec((_R, D), lambda i: (i, 0)),
            pl.BlockSpec((D, H), lambda i: (0, 0)),
            pl.BlockSpec((_R, NW), lambda i: (i, 0)),
        ],
        out_specs=[
            pl.BlockSpec((_R, H), lambda i: (i, 0)),
            pl.BlockSpec((_R, 1), lambda i: (i, 0)),
        ],
        out_shape=[
            jax.ShapeDtypeStruct((NP, H), jnp.float32),
            jax.ShapeDtypeStruct((NP, 1), jnp.float32),
        ],
    )(xp, W1, degt)

    part1 = _agg1_kernel(hs1, src3, dst3)

    hs2 = pl.pallas_call(
        _tc2_body,
        grid=(NP // _R,),
        in_specs=[
            pl.BlockSpec((NC, _R, H), lambda i: (0, i, 0)),
            pl.BlockSpec((_R, H), lambda i: (i, 0)),
            pl.BlockSpec((_R, 1), lambda i: (i, 0)),
            pl.BlockSpec((H, 1), lambda i: (0, 0)),
            pl.BlockSpec((1, H), lambda i: (0, 0)),
        ],
        out_specs=pl.BlockSpec((_R, 1), lambda i: (i, 0)),
        out_shape=jax.ShapeDtypeStruct((NP, 1), jnp.float32),
    )(part1, hs1, dis, W2, b1.reshape(1, H))

    part2 = _agg2_kernel(hs2.reshape(NP), src_w, dst_w)
    part2t = part2.T

    out = pl.pallas_call(
        _tc3_body,
        grid=(NP // _R,),
        in_specs=[
            pl.BlockSpec((_R, NW), lambda i: (i, 0)),
            pl.BlockSpec((_R, 1), lambda i: (i, 0)),
            pl.BlockSpec((_R, 1), lambda i: (i, 0)),
            pl.BlockSpec((1, 1), lambda i: (0, 0)),
        ],
        out_specs=pl.BlockSpec((_R, 1), lambda i: (i, 0)),
        out_shape=jax.ShapeDtypeStruct((NP, 1), jnp.float32),
    )(part2t, hs2, dis, b2.reshape(1, 1))

    return out[:N]
